# Initial kernel scaffold; baseline (speedup 1.0000x reference)
import jax
import jax.numpy as jnp
from jax.experimental import pallas as pl


def kernel(feat_lr, guide_hr, sx_raw, sy_raw, th_raw, sr_raw):
    raise NotImplementedError("write your pallas kernel here")


# speedup vs baseline: 293.4357x; 293.4357x over previous
import math

import jax
import jax.numpy as jnp
from jax.experimental import pallas as pl
from jax.experimental.pallas import tpu as pltpu

SCALE = 16
HL, WL = 14, 14
CF = 96
HH, WH = 224, 224
_TAPS = [(dy, dx) for dy in range(-2, 3) for dx in range(-2, 3)
         if dy * dy + dx * dx <= 4]
_NT = len(_TAPS)


def _shift_cols(x, dx):
  if dx == 0:
    return x
  if dx > 0:
    return jnp.concatenate([x[:, dx:]] + [x[:, -1:]] * dx, axis=1)
  return jnp.concatenate([x[:, :1]] * (-dx) + [x[:, :WL + dx]], axis=1)


def _up16(x):
  return jnp.repeat(x, SCALE, axis=1)


def _body(feat_ref, gstrip_ref, grow_ref, par_ref, out_ref, fup_ref, w_ref):
  u = pl.program_id(0)
  f32 = jnp.float32
  u_f = u.astype(f32)

  x_i = jax.lax.broadcasted_iota(jnp.int32, (1, WH), 1)
  x_f = x_i.astype(f32)
  jj = jax.lax.broadcasted_iota(jnp.int32, (WL, WH), 0)
  xx = jax.lax.broadcasted_iota(jnp.int32, (WL, WH), 1)
  m7 = (xx == jj * SCALE + 7).astype(f32)
  m8 = (xx == jj * SCALE + 8).astype(f32)

  p = par_ref[:, pl.ds(u, 1), :, :].reshape(4, WL)
  sx = jnp.maximum(jnp.exp(p[0:1]), 1e-6)
  sy = jnp.maximum(jnp.exp(p[1:2]), 1e-6)
  th = math.pi * jnp.tanh(p[2:3])
  sr = jnp.maximum(jnp.exp(p[3:4]), 1e-6)
  D = jnp.concatenate([
      jnp.cos(th), jnp.sin(th),
      2.0 * sx ** 2 + 1e-8,
      2.0 * sy ** 2 + 1e-8,
      2.0 * sr ** 2 + 1e-8,
      jnp.clip(2.0 * jnp.maximum(sx, sy), 1.0, 2.0) ** 2,
  ], axis=0)
  Dup = _up16(D)
  cos_up, sin_up = Dup[0:1], Dup[1:2]
  d1_up, d2_up = Dup[2:3], Dup[3:4]
  d3_up, rsq_up = Dup[4:5], Dup[5:6]

  gs = gstrip_ref[...]

  glr = {}
  frow = {}
  uis = {}
  for dy in range(-2, 3):
    ui = jnp.clip(u + dy, 0, HL - 1)
    uis[dy] = ui
    r2 = grow_ref[:, pl.ds(ui, 1), pl.ds(7, 2), :]
    row7 = r2[:, 0, 0, :][:, None, :]
    row8 = r2[:, 0, 1, :][:, None, :]
    v00 = jnp.sum(row7 * m7[None], axis=2)
    v01 = jnp.sum(row7 * m8[None], axis=2)
    v10 = jnp.sum(row8 * m7[None], axis=2)
    v11 = jnp.sum(row8 * m8[None], axis=2)
    glr[dy] = 0.25 * (((v00 + v01) + v10) + v11)
    frow[dy] = feat_ref[pl.ds(ui, 1), :, :].reshape(CF, WL)

  riota = jax.lax.broadcasted_iota(jnp.int32, (SCALE, 1), 0).astype(f32)
  den = jnp.zeros((SCALE, WH), f32)

  for ti, (dy, dx) in enumerate(_TAPS):
    ui_f = uis[dy].astype(f32)
    vi_x = jnp.clip(x_i // SCALE + dx, 0, WL - 1).astype(f32)
    cur_dx = (x_f - (vi_x * SCALE + (SCALE - 1) / 2.0)) / SCALE
    cur_dy = (u_f - ui_f) + (riota - (SCALE - 1) / 2.0) / SCALE
    a = cur_dx * cos_up + cur_dy * sin_up
    b = (-cur_dx) * sin_up + cur_dy * cos_up
    logw = (-(a * a)) / d1_up - (b * b) / d2_up
    gup = _up16(_shift_cols(glr[dy], dx))
    gd = ((gs[0] - gup[0:1]) ** 2 + (gs[1] - gup[1:2]) ** 2
          + (gs[2] - gup[2:3]) ** 2)
    logw = logw - gd / d3_up
    w = jnp.exp(logw)
    c2 = float(dy * dy + dx * dx)
    if c2 > 1.0:
      w = w * (c2 <= rsq_up).astype(f32)
    den = den + w
    w_ref[ti] = w
    fup_ref[ti] = _up16(_shift_cols(frow[dy], dx))

  denc = jnp.maximum(den, 1e-8)
  for r in range(SCALE):
    acc = fup_ref[0] * w_ref[0, r, :]
    for ti in range(1, _NT):
      acc = acc + fup_ref[ti] * w_ref[ti, r, :]
    out_ref[:, r, :] = acc / denc[r, :]


def kernel(feat_lr, guide_hr, sx_raw, sy_raw, th_raw, sr_raw):
  f32 = jnp.float32
  feat_t = jnp.transpose(feat_lr[0].astype(f32), (1, 0, 2))
  guide = guide_hr[0].astype(f32)
  guide4 = guide.reshape(3, HL, SCALE, WH)
  par = jnp.concatenate([
      sx_raw, sy_raw, th_raw, sr_raw], axis=1)[0].astype(f32)
  par = par.reshape(4, HL, 1, WL)

  out = pl.pallas_call(
      _body,
      grid=(HL,),
      in_specs=[
          pl.BlockSpec((HL, CF, WL), lambda u: (0, 0, 0)),
          pl.BlockSpec((3, SCALE, WH), lambda u: (0, u, 0)),
          pl.BlockSpec((3, HL, SCALE, WH), lambda u: (0, 0, 0, 0)),
          pl.BlockSpec((4, HL, 1, WL), lambda u: (0, 0, 0, 0)),
      ],
      out_specs=pl.BlockSpec((CF, SCALE, WH), lambda u: (0, u, 0)),
      out_shape=jax.ShapeDtypeStruct((CF, HH, WH), f32),
      scratch_shapes=[
          pltpu.VMEM((_NT, CF, WH), f32),
          pltpu.VMEM((_NT, SCALE, WH), f32),
      ],
  )(feat_t, guide, guide4, par)
  return out[None].astype(feat_lr.dtype)


# speedup vs baseline: 485.1245x; 1.6533x over previous
import math

import jax
import jax.numpy as jnp
from jax.experimental import pallas as pl
from jax.experimental.pallas import tpu as pltpu

SCALE = 16
HL, WL = 14, 14
CF = 96
HH, WH = 224, 224
_TAPS = [(dy, dx) for dy in range(-2, 3) for dx in range(-2, 3)
         if dy * dy + dx * dx <= 4]
_NT = len(_TAPS)


def _up16(x):
  return jnp.repeat(x, SCALE, axis=1)


def _shift_up(xu, dx):
  s = SCALE * dx
  if dx == 0:
    return xu
  if dx > 0:
    return jnp.concatenate(
        [xu[:, s:]] + [xu[:, WH - SCALE:]] * dx, axis=1)
  return jnp.concatenate(
      [xu[:, :SCALE]] * (-dx) + [xu[:, :WH + s]], axis=1)


def _body(feat_ref, gstrip_ref, grow_ref, par_ref, out_ref, fup_ref, w_ref):
  u = pl.program_id(0)
  f32 = jnp.float32
  u_f = u.astype(f32)

  x_i = jax.lax.broadcasted_iota(jnp.int32, (1, WH), 1)
  x_f = x_i.astype(f32)
  jj = jax.lax.broadcasted_iota(jnp.int32, (WL, WH), 0)
  xx = jax.lax.broadcasted_iota(jnp.int32, (WL, WH), 1)
  m7 = (xx == jj * SCALE + 7).astype(f32)
  m8 = (xx == jj * SCALE + 8).astype(f32)

  p = par_ref[:, pl.ds(u, 1), :, :].reshape(4, WL)
  sx = jnp.maximum(jnp.exp(p[0:1]), 1e-6)
  sy = jnp.maximum(jnp.exp(p[1:2]), 1e-6)
  th = math.pi * jnp.tanh(p[2:3])
  sr = jnp.maximum(jnp.exp(p[3:4]), 1e-6)
  D = jnp.concatenate([
      jnp.cos(th), jnp.sin(th),
      2.0 * sx ** 2 + 1e-8,
      2.0 * sy ** 2 + 1e-8,
      2.0 * sr ** 2 + 1e-8,
      jnp.clip(2.0 * jnp.maximum(sx, sy), 1.0, 2.0) ** 2,
  ], axis=0)
  Dup = _up16(D)
  cos_up, sin_up = Dup[0:1], Dup[1:2]
  d1_up, d2_up = Dup[2:3], Dup[3:4]
  d3_up, rsq_up = Dup[4:5], Dup[5:6]

  gs = gstrip_ref[...]

  riota = jax.lax.broadcasted_iota(jnp.int32, (SCALE, 1), 0).astype(f32)
  den = jnp.zeros((SCALE, WH), f32)

  ti = 0
  for dy in range(-2, 3):
    ui = jnp.clip(u + dy, 0, HL - 1)
    ui_f = ui.astype(f32)
    r2 = grow_ref[:, pl.ds(ui, 1), pl.ds(7, 2), :]
    row7 = r2[:, 0, 0, :][:, None, :]
    row8 = r2[:, 0, 1, :][:, None, :]
    v00 = jnp.sum(row7 * m7[None], axis=2)
    v01 = jnp.sum(row7 * m8[None], axis=2)
    v10 = jnp.sum(row8 * m7[None], axis=2)
    v11 = jnp.sum(row8 * m8[None], axis=2)
    gbase = _up16(0.25 * (((v00 + v01) + v10) + v11))
    frow = feat_ref[pl.ds(ui, 1), :, :].reshape(CF, WL)
    fbase = _up16(frow)
    cur_dy = (u_f - ui_f) + (riota - (SCALE - 1) / 2.0) / SCALE

    for dx in range(-2, 3):
      c2 = float(dy * dy + dx * dx)
      if c2 > 4.0:
        continue
      vi_x = jnp.clip(x_i // SCALE + dx, 0, WL - 1).astype(f32)
      cur_dx = (x_f - (vi_x * SCALE + (SCALE - 1) / 2.0)) / SCALE
      a = cur_dx * cos_up + cur_dy * sin_up
      b = (-cur_dx) * sin_up + cur_dy * cos_up
      logw = (-(a * a)) / d1_up - (b * b) / d2_up
      gup = _shift_up(gbase, dx)
      gd = ((gs[0] - gup[0:1]) ** 2 + (gs[1] - gup[1:2]) ** 2
            + (gs[2] - gup[2:3]) ** 2)
      logw = logw - gd / d3_up
      w = jnp.exp(logw)
      if c2 > 1.0:
        w = w * (c2 <= rsq_up).astype(f32)
      den = den + w
      w_ref[ti] = w
      fup_ref[ti] = _shift_up(fbase, dx)
      ti += 1
  assert ti == _NT

  denc = jnp.maximum(den, 1e-8)
  for r in range(SCALE):
    acc = fup_ref[0] * w_ref[0, r, :]
    for ti in range(1, _NT):
      acc = acc + fup_ref[ti] * w_ref[ti, r, :]
    out_ref[:, r, :] = acc / denc[r, :]


def kernel(feat_lr, guide_hr, sx_raw, sy_raw, th_raw, sr_raw):
  f32 = jnp.float32
  feat_t = jnp.transpose(feat_lr[0].astype(f32), (1, 0, 2))
  guide = guide_hr[0].astype(f32)
  guide4 = guide.reshape(3, HL, SCALE, WH)
  par = jnp.concatenate([
      sx_raw, sy_raw, th_raw, sr_raw], axis=1)[0].astype(f32)
  par = par.reshape(4, HL, 1, WL)

  out = pl.pallas_call(
      _body,
      grid=(HL,),
      in_specs=[
          pl.BlockSpec((HL, CF, WL), lambda u: (0, 0, 0)),
          pl.BlockSpec((3, SCALE, WH), lambda u: (0, u, 0)),
          pl.BlockSpec((3, HL, SCALE, WH), lambda u: (0, 0, 0, 0)),
          pl.BlockSpec((4, HL, 1, WL), lambda u: (0, 0, 0, 0)),
      ],
      out_specs=pl.BlockSpec((CF, SCALE, WH), lambda u: (0, u, 0)),
      out_shape=jax.ShapeDtypeStruct((CF, HH, WH), f32),
      scratch_shapes=[
          pltpu.VMEM((_NT, CF, WH), f32),
          pltpu.VMEM((_NT, SCALE, WH), f32),
      ],
  )(feat_t, guide, guide4, par)
  return out[None].astype(feat_lr.dtype)


# speedup vs baseline: 711.3822x; 1.4664x over previous
import math

import jax
import jax.numpy as jnp
from jax.experimental import pallas as pl
from jax.experimental.pallas import tpu as pltpu

SCALE = 16
HL, WL = 14, 14
CF = 96
HH, WH = 224, 224
_TAPS = [(dy, dx) for dy in range(-2, 3) for dx in range(-2, 3)
         if dy * dy + dx * dx <= 4]
_NT = len(_TAPS)


def _up16(x):
  return jnp.repeat(x, SCALE, axis=1)


def _shift_up(xu, dx):
  s = SCALE * dx
  if dx == 0:
    return xu
  if dx > 0:
    return jnp.concatenate(
        [xu[:, s:]] + [xu[:, WH - SCALE:]] * dx, axis=1)
  return jnp.concatenate(
      [xu[:, :SCALE]] * (-dx) + [xu[:, :WH + s]], axis=1)


def _body(feat_ref, gstrip_ref, grow_ref, par_ref, out_ref, fup_ref, w_ref,
          fball_ref, gball_ref):
  u = pl.program_id(0)
  f32 = jnp.float32
  u_f = u.astype(f32)

  x_i = jax.lax.broadcasted_iota(jnp.int32, (1, WH), 1)
  x_f = x_i.astype(f32)
  jj = jax.lax.broadcasted_iota(jnp.int32, (WL, WH), 0)
  xx = jax.lax.broadcasted_iota(jnp.int32, (WL, WH), 1)
  m7 = (xx == jj * SCALE + 7).astype(f32)
  m8 = (xx == jj * SCALE + 8).astype(f32)

  @pl.when(u == 0)
  def _precompute():
    for i in range(HL):
      fball_ref[i] = _up16(feat_ref[i])
      r2 = grow_ref[:, pl.ds(i, 1), pl.ds(7, 2), :]
      row7 = r2[:, 0, 0, :][:, None, :]
      row8 = r2[:, 0, 1, :][:, None, :]
      v00 = jnp.sum(row7 * m7[None], axis=2)
      v01 = jnp.sum(row7 * m8[None], axis=2)
      v10 = jnp.sum(row8 * m7[None], axis=2)
      v11 = jnp.sum(row8 * m8[None], axis=2)
      gball_ref[i] = _up16(0.25 * (((v00 + v01) + v10) + v11))

  p = par_ref[:, pl.ds(u, 1), :, :].reshape(4, WL)
  sx = jnp.maximum(jnp.exp(p[0:1]), 1e-6)
  sy = jnp.maximum(jnp.exp(p[1:2]), 1e-6)
  th = math.pi * jnp.tanh(p[2:3])
  sr = jnp.maximum(jnp.exp(p[3:4]), 1e-6)
  D = jnp.concatenate([
      jnp.cos(th), jnp.sin(th),
      2.0 * sx ** 2 + 1e-8,
      2.0 * sy ** 2 + 1e-8,
      2.0 * sr ** 2 + 1e-8,
      jnp.clip(2.0 * jnp.maximum(sx, sy), 1.0, 2.0) ** 2,
  ], axis=0)
  Dup = _up16(D)
  cos_up, sin_up = Dup[0:1], Dup[1:2]
  d1_up, d2_up = Dup[2:3], Dup[3:4]
  d3_up, rsq_up = Dup[4:5], Dup[5:6]

  gs = gstrip_ref[...]

  riota = jax.lax.broadcasted_iota(jnp.int32, (SCALE, 1), 0).astype(f32)
  den = jnp.zeros((SCALE, WH), f32)

  ti = 0
  for dy in range(-2, 3):
    ui = jnp.clip(u + dy, 0, HL - 1)
    ui_f = ui.astype(f32)
    gbase = gball_ref[pl.ds(ui, 1), :, :].reshape(3, WH)
    fbase = fball_ref[pl.ds(ui, 1), :, :].reshape(CF, WH)
    cur_dy = (u_f - ui_f) + (riota - (SCALE - 1) / 2.0) / SCALE

    for dx in range(-2, 3):
      c2 = float(dy * dy + dx * dx)
      if c2 > 4.0:
        continue
      vi_x = jnp.clip(x_i // SCALE + dx, 0, WL - 1).astype(f32)
      cur_dx = (x_f - (vi_x * SCALE + (SCALE - 1) / 2.0)) / SCALE
      a = cur_dx * cos_up + cur_dy * sin_up
      b = (-cur_dx) * sin_up + cur_dy * cos_up
      logw = (-(a * a)) / d1_up - (b * b) / d2_up
      gup = _shift_up(gbase, dx)
      gd = ((gs[0] - gup[0:1]) ** 2 + (gs[1] - gup[1:2]) ** 2
            + (gs[2] - gup[2:3]) ** 2)
      logw = logw - gd / d3_up
      w = jnp.exp(logw)
      if c2 > 1.0:
        w = w * (c2 <= rsq_up).astype(f32)
      den = den + w
      w_ref[ti] = w
      fup_ref[ti] = _shift_up(fbase, dx)
      ti += 1
  assert ti == _NT

  invd = 1.0 / jnp.maximum(den, 1e-8)
  for r in range(SCALE):
    acc = fup_ref[0] * w_ref[0, r, :]
    for ti in range(1, _NT):
      acc = acc + fup_ref[ti] * w_ref[ti, r, :]
    out_ref[:, r, :] = acc * invd[r, :]


def kernel(feat_lr, guide_hr, sx_raw, sy_raw, th_raw, sr_raw):
  f32 = jnp.float32
  feat_t = jnp.transpose(feat_lr[0].astype(f32), (1, 0, 2))
  guide = guide_hr[0].astype(f32)
  guide4 = guide.reshape(3, HL, SCALE, WH)
  par = jnp.concatenate([
      sx_raw, sy_raw, th_raw, sr_raw], axis=1)[0].astype(f32)
  par = par.reshape(4, HL, 1, WL)

  out = pl.pallas_call(
      _body,
      grid=(HL,),
      in_specs=[
          pl.BlockSpec((HL, CF, WL), lambda u: (0, 0, 0)),
          pl.BlockSpec((3, SCALE, WH), lambda u: (0, u, 0)),
          pl.BlockSpec((3, HL, SCALE, WH), lambda u: (0, 0, 0, 0)),
          pl.BlockSpec((4, HL, 1, WL), lambda u: (0, 0, 0, 0)),
      ],
      out_specs=pl.BlockSpec((CF, SCALE, WH), lambda u: (0, u, 0)),
      out_shape=jax.ShapeDtypeStruct((CF, HH, WH), f32),
      scratch_shapes=[
          pltpu.VMEM((_NT, CF, WH), f32),
          pltpu.VMEM((_NT, SCALE, WH), f32),
          pltpu.VMEM((HL, CF, WH), f32),
          pltpu.VMEM((HL, 3, WH), f32),
      ],
  )(feat_t, guide, guide4, par)
  return out[None].astype(feat_lr.dtype)
